# baseline (speedup 1.0000x reference)
import jax
import jax.numpy as jnp
from jax.experimental import pallas as pl

_CHUNK = 64


def _bf16r(x):
    return x.astype(jnp.bfloat16).astype(jnp.float32)


def _diffnet_kernel(x_ref, at_ref, th_ref, ph_ref, w1_ref, b1_ref,
                    w2_ref, b2_ref, out_ref):
    B, N, F = x_ref.shape
    C1 = th_ref.shape[0]
    C2 = ph_ref.shape[0]

    pooled_cols = []
    for b in range(B):
        at = at_ref[b]
        sq = jnp.sum(_bf16r(x_ref[b]), axis=1, keepdims=True)

        arg_chunks = []
        for c0 in range(0, C1, _CHUNK):
            th0 = th_ref[c0:c0 + _CHUNK, :, 0:1]
            th1 = th_ref[c0:c0 + _CHUNK, :, 1:2]
            th2 = th_ref[c0:c0 + _CHUNK, :, 2:3]
            p = (th0 * at[None, :, :] + th1) * at[None, :, :] + th2
            pq = _bf16r(p)
            arg_chunks.append(jnp.sum(pq * sq[None, :, :], axis=1))
        h1 = jnp.tanh(jnp.concatenate(arg_chunks, axis=0))

        s2_row = jnp.sum(_bf16r(h1), axis=0, keepdims=True)
        s2 = jax.lax.transpose(s2_row, (1, 0))

        arg2_chunks = []
        for c0 in range(0, C2, _CHUNK):
            ph0 = ph_ref[c0:c0 + _CHUNK, :, 0:1]
            ph1 = ph_ref[c0:c0 + _CHUNK, :, 1:2]
            pq2 = _bf16r(ph0 * at[None, :, :] + ph1)
            arg2_chunks.append(jnp.sum(pq2 * s2[None, :, :], axis=1))
        h2 = jnp.tanh(jnp.concatenate(arg2_chunks, axis=0))

        pooled_cols.append(jnp.sum(h2, axis=1, keepdims=True))

    pooled = jax.lax.transpose(jnp.concatenate(pooled_cols, axis=1), (1, 0))

    d1 = jnp.tanh(
        jax.lax.dot(pooled, w1_ref[...], preferred_element_type=jnp.float32)
        + b1_ref[...])
    logits = (jax.lax.dot(d1, w2_ref[...], preferred_element_type=jnp.float32)
              + b2_ref[...])
    m = jnp.max(logits, axis=1, keepdims=True)
    e = jnp.exp(logits - m)
    out_ref[...] = e / jnp.sum(e, axis=1, keepdims=True)


def kernel(x_batch, adj, k1, k2, Wd1, bd1, Wd2, bd2):
    B, N, F = x_batch.shape
    adjT = adj.swapaxes(1, 2)
    th = k1.reshape(k1.shape[0], 1, 3)
    ph = k2.reshape(k2.shape[0], 1, 2)
    b1 = bd1.reshape(1, -1)
    b2 = bd2.reshape(1, -1)

    return pl.pallas_call(
        _diffnet_kernel,
        out_shape=jax.ShapeDtypeStruct((B, 10), jnp.float32),
    )(x_batch, adjT, th, ph, Wd1, b1, Wd2, b2)


# speedup vs baseline: 1.2440x; 1.2440x over previous
import jax
import jax.numpy as jnp
from jax.experimental import pallas as pl
from jax.experimental.pallas import tpu as pltpu

_CHUNK = 64


def _bf16r(x):
    return x.astype(jnp.bfloat16).astype(jnp.float32)


def _pool_kernel(x_ref, at_ref, th_ref, ph_ref, pooled_ref):
    C1 = th_ref.shape[0]
    C2 = ph_ref.shape[0]
    at = at_ref[0]
    sq = jnp.sum(_bf16r(x_ref[0]), axis=1, keepdims=True)

    arg_chunks = []
    for c0 in range(0, C1, _CHUNK):
        th0 = th_ref[c0:c0 + _CHUNK, :, 0:1]
        th1 = th_ref[c0:c0 + _CHUNK, :, 1:2]
        th2 = th_ref[c0:c0 + _CHUNK, :, 2:3]
        p = (th0 * at[None, :, :] + th1) * at[None, :, :] + th2
        pq = _bf16r(p)
        arg_chunks.append(jnp.sum(pq * sq[None, :, :], axis=1))
    h1 = jnp.tanh(jnp.concatenate(arg_chunks, axis=0))

    s2_row = jnp.sum(_bf16r(h1), axis=0, keepdims=True)
    s2 = jax.lax.transpose(s2_row, (1, 0))

    arg2_chunks = []
    for c0 in range(0, C2, _CHUNK):
        ph0 = ph_ref[c0:c0 + _CHUNK, :, 0:1]
        ph1 = ph_ref[c0:c0 + _CHUNK, :, 1:2]
        pq2 = _bf16r(ph0 * at[None, :, :] + ph1)
        arg2_chunks.append(jnp.sum(pq2 * s2[None, :, :], axis=1))
    h2 = jnp.tanh(jnp.concatenate(arg2_chunks, axis=0))

    pooled_ref[0] = jnp.sum(h2, axis=1, keepdims=True)


def _head_kernel(pooled_ref, w1_ref, b1_ref, w2_ref, b2_ref, out_ref):
    pooled = pooled_ref[...][:, :, 0]
    d1 = jnp.tanh(
        jax.lax.dot(pooled, w1_ref[...], preferred_element_type=jnp.float32)
        + b1_ref[...])
    logits = (jax.lax.dot(d1, w2_ref[...], preferred_element_type=jnp.float32)
              + b2_ref[...])
    m = jnp.max(logits, axis=1, keepdims=True)
    e = jnp.exp(logits - m)
    out_ref[...] = e / jnp.sum(e, axis=1, keepdims=True)


def kernel(x_batch, adj, k1, k2, Wd1, bd1, Wd2, bd2):
    B, N, F = x_batch.shape
    C2 = k2.shape[0]
    adjT = adj.swapaxes(1, 2)
    th = k1.reshape(k1.shape[0], 1, 3)
    ph = k2.reshape(k2.shape[0], 1, 2)
    b1 = bd1.reshape(1, -1)
    b2 = bd2.reshape(1, -1)

    pooled = pl.pallas_call(
        _pool_kernel,
        grid=(B,),
        in_specs=[
            pl.BlockSpec((1, N, F), lambda b: (b, 0, 0)),
            pl.BlockSpec((1, N, N), lambda b: (b, 0, 0)),
            pl.BlockSpec(th.shape, lambda b: (0, 0, 0)),
            pl.BlockSpec(ph.shape, lambda b: (0, 0, 0)),
        ],
        out_specs=pl.BlockSpec((1, C2, 1), lambda b: (b, 0, 0)),
        out_shape=jax.ShapeDtypeStruct((B, C2, 1), jnp.float32),
        compiler_params=pltpu.CompilerParams(
            dimension_semantics=("parallel",)),
    )(x_batch, adjT, th, ph)

    return pl.pallas_call(
        _head_kernel,
        out_shape=jax.ShapeDtypeStruct((B, 10), jnp.float32),
    )(pooled, Wd1, b1, Wd2, b2)
